# baseline (speedup 1.0000x reference)
import jax
import jax.numpy as jnp
from jax.experimental import pallas as pl
from jax.experimental.pallas import tpu as pltpu

_CH = 10
_ROWS = 1000


def _mm_body(x_hbm, w_ref, b_ref, o_hbm, xbuf, obuf, in_sems, out_sems):
    copies_in = []
    for i in range(_CH):
        c = pltpu.make_async_copy(
            x_hbm.at[pl.ds(i * _ROWS, _ROWS), :], xbuf.at[i], in_sems.at[i])
        c.start()
        copies_in.append(c)
    w = w_ref[...]
    b = b_ref[...]
    copies_out = []
    for i in range(_CH):
        copies_in[i].wait()
        x = jnp.maximum(xbuf[i], 0.0)
        obuf[i] = jax.lax.dot_general(
            x, w, (((1,), (0,)), ((), ())),
            preferred_element_type=jnp.float32) + b
        c = pltpu.make_async_copy(
            obuf.at[i], o_hbm.at[pl.ds(i * _ROWS, _ROWS), :], out_sems.at[i])
        c.start()
        copies_out.append(c)
    for c in copies_out:
        c.wait()


def kernel(x_subject, x_region, edge_index_sr, edge_index_rr, edge_attr_sr,
           edge_attr_rr, sage_Wl0, sage_bl0, sage_Wr0, gcn_W0, gcn_b0,
           sage_Wl1, sage_bl1, sage_Wr1, gcn_W1, gcn_b1, lin_W, lin_b):
    m, d = x_subject.shape
    out_dim = lin_W.shape[1]
    return pl.pallas_call(
        _mm_body,
        in_specs=[
            pl.BlockSpec(memory_space=pltpu.MemorySpace.HBM),
            pl.BlockSpec(memory_space=pltpu.MemorySpace.VMEM),
            pl.BlockSpec(memory_space=pltpu.MemorySpace.VMEM),
        ],
        out_specs=pl.BlockSpec(memory_space=pltpu.MemorySpace.HBM),
        out_shape=jax.ShapeDtypeStruct((m, out_dim), jnp.float32),
        scratch_shapes=[
            pltpu.VMEM((_CH, _ROWS, d), jnp.float32),
            pltpu.VMEM((_CH, _ROWS, out_dim), jnp.float32),
            pltpu.SemaphoreType.DMA((_CH,)),
            pltpu.SemaphoreType.DMA((_CH,)),
        ],
    )(x_subject, lin_W, lin_b.reshape(1, out_dim))
